# baseline (speedup 1.0000x reference)
import functools

import jax
import jax.numpy as jnp
from jax import lax
from jax.experimental import pallas as pl
from jax.experimental.pallas import tpu as pltpu
from jax.experimental.pallas import tpu_sc as plsc

N_NODES = 10000
N_EDGES = 320000
D = 128

NC = 2
NS = 16
NW = NC * NS
CHUNK = 50
CPW = N_EDGES // NW // CHUNK
NPHASE = 5
CPP = CPW // NPHASE
ROWS_PER_TILE = 632
ACC_ROWS = NS * ROWS_PER_TILE
ZCH = 40

_mesh = plsc.VectorSubcoreMesh(core_axis_name="c", subcore_axis_name="s")


@functools.partial(
    pl.kernel,
    out_type=jax.ShapeDtypeStruct((NC, ACC_ROWS, D), jnp.float32),
    mesh=_mesh,
    scratch_types=[
        pltpu.VMEM((CPP, CHUNK), jnp.int32),
        pltpu.VMEM((CPP, CHUNK), jnp.int32),
        pltpu.VMEM((CHUNK, D), jnp.float32),
        pltpu.VMEM((CHUNK, D), jnp.float32),
        pltpu.VMEM_SHARED((ACC_ROWS, D), jnp.float32),
        pltpu.SemaphoreType.DMA,
        pltpu.SemaphoreType.DMA,
    ],
)
def _sc_aggregate(neigh_hbm, eidx_hbm, out_hbm,
                  row_v, col_v, buf0, buf1, acc, sem0, sem1):
    cid = lax.axis_index("c")
    sid = lax.axis_index("s")
    wid = cid * NS + sid

    zero16 = jnp.zeros((16,), jnp.float32)

    @pl.loop(0, CHUNK)
    def _zero_rows(r):
        for j in range(D // 16):
            buf0[r, pl.ds(j * 16, 16)] = zero16

    stripe = sid * ROWS_PER_TILE
    for t in range(ROWS_PER_TILE // ZCH):
        pltpu.sync_copy(buf0.at[pl.ds(0, ZCH)],
                        acc.at[pl.ds(stripe + t * ZCH, ZCH)])
    rem = ROWS_PER_TILE % ZCH
    pltpu.sync_copy(buf0.at[pl.ds(0, rem)],
                    acc.at[pl.ds(stripe + ROWS_PER_TILE - rem, rem)])

    plsc.subcore_barrier()

    bufs = (buf0, buf1)
    sems = (sem0, sem1)

    for phase in range(NPHASE):
        base = wid * CPW + phase * CPP
        pltpu.sync_copy(eidx_hbm.at[0, pl.ds(base, CPP)], row_v)
        pltpu.sync_copy(eidx_hbm.at[1, pl.ds(base, CPP)], col_v)

        pltpu.async_copy(neigh_hbm.at[row_v.at[0]], buf0, sem0)

        @pl.loop(0, CPP, step=2)
        def _chunks(g):
            for b in range(2):
                j = g + b
                @pl.when(j + 1 < CPP)
                def _():
                    pltpu.async_copy(
                        neigh_hbm.at[row_v.at[j + 1]], bufs[1 - b], sems[1 - b])
                pltpu.make_async_copy(
                    neigh_hbm.at[row_v.at[j]], bufs[b], sems[b]).wait()
                pltpu.sync_copy(bufs[b], acc.at[col_v.at[j]], add=True)

    plsc.subcore_barrier()

    pltpu.sync_copy(acc.at[pl.ds(stripe, ROWS_PER_TILE)],
                    out_hbm.at[cid, pl.ds(stripe, ROWS_PER_TILE)])


_BLK = 1000
_DN = (((1,), (1,)), ((), ()))


def _mm_body(x_ref, w_ref, b_ref, o_ref):
    o_ref[...] = lax.dot_general(
        x_ref[...], w_ref[...], _DN, preferred_element_type=jnp.float32
    ) + b_ref[...]


def _addrelu_body(selfp_ref, p0_ref, p1_ref, o_ref):
    o_ref[...] = jnp.maximum(
        selfp_ref[...] + p0_ref[0, :, :] + p1_ref[0, :, :], 0.0)


def _matmul_bias(x, W, b):
    nblk = N_NODES // _BLK
    return pl.pallas_call(
        _mm_body,
        grid=(nblk,),
        in_specs=[
            pl.BlockSpec((_BLK, D), lambda i: (i, 0)),
            pl.BlockSpec((D, D), lambda i: (0, 0)),
            pl.BlockSpec((1, D), lambda i: (0, 0)),
        ],
        out_specs=pl.BlockSpec((_BLK, D), lambda i: (i, 0)),
        out_shape=jax.ShapeDtypeStruct((N_NODES, D), jnp.float32),
    )(x, W, b.reshape(1, D))


def kernel(x, edge_index, W_self, b_self, W_neigh, b_neigh):
    eidx = edge_index.astype(jnp.int32).reshape(2, NW * CPW, CHUNK)

    neigh = _matmul_bias(x, W_neigh, b_neigh)
    partials = _sc_aggregate(neigh, eidx)
    selfp = _matmul_bias(x, W_self, b_self)

    nblk = N_NODES // _BLK
    out = pl.pallas_call(
        _addrelu_body,
        grid=(nblk,),
        in_specs=[
            pl.BlockSpec((_BLK, D), lambda i: (i, 0)),
            pl.BlockSpec((1, _BLK, D), lambda i: (0, i, 0)),
            pl.BlockSpec((1, _BLK, D), lambda i: (1, i, 0)),
        ],
        out_specs=pl.BlockSpec((_BLK, D), lambda i: (i, 0)),
        out_shape=jax.ShapeDtypeStruct((N_NODES, D), jnp.float32),
    )(selfp, partials, partials)

    return out


# speedup vs baseline: 1.0859x; 1.0859x over previous
import functools

import jax
import jax.numpy as jnp
from jax import lax
from jax.experimental import pallas as pl
from jax.experimental.pallas import tpu as pltpu
from jax.experimental.pallas import tpu_sc as plsc

N_NODES = 10000
N_EDGES = 320000
D = 128

NC = 2
NS = 16
NW = NC * NS
EPW = N_EDGES // NW
CHUNK = 64
NPHASE = 3
CPP = 52
EPP = CPP * CHUNK
TAIL = EPW - NPHASE * EPP
ROWS_PER_TILE = 632
ACC_ROWS = NS * ROWS_PER_TILE
ZCH = 40

_mesh = plsc.VectorSubcoreMesh(core_axis_name="c", subcore_axis_name="s")


@functools.partial(
    pl.kernel,
    out_type=jax.ShapeDtypeStruct((NC, ACC_ROWS, D), jnp.float32),
    mesh=_mesh,
    scratch_types=[
        pltpu.VMEM((EPP,), jnp.int32),
        pltpu.VMEM((EPP,), jnp.int32),
        pltpu.VMEM((1, CHUNK), jnp.int32),
        pltpu.VMEM((1, CHUNK), jnp.int32),
        pltpu.VMEM((1, 16), jnp.int32),
        pltpu.VMEM((CHUNK, D), jnp.float32),
        pltpu.VMEM((CHUNK, D), jnp.float32),
        pltpu.VMEM_SHARED((ACC_ROWS, D), jnp.float32),
        pltpu.SemaphoreType.DMA,
        pltpu.SemaphoreType.DMA,
    ],
)
def _sc_aggregate(neigh_hbm, rows_hbm, cols_hbm, out_hbm,
                  row_v, col_v, cidx0, cidx1, tidx, buf0, buf1, acc,
                  sem0, sem1):
    cid = lax.axis_index("c")
    sid = lax.axis_index("s")
    wid = cid * NS + sid
    ebase = wid * EPW

    zero16 = jnp.zeros((16,), jnp.float32)

    @pl.loop(0, CHUNK)
    def _zero_rows(r):
        for j in range(D // 16):
            buf0[r, pl.ds(j * 16, 16)] = zero16

    stripe = sid * ROWS_PER_TILE
    for t in range(ROWS_PER_TILE // ZCH):
        pltpu.sync_copy(buf0.at[pl.ds(0, ZCH)],
                        acc.at[pl.ds(stripe + t * ZCH, ZCH)])
    rem = ROWS_PER_TILE % ZCH
    pltpu.sync_copy(buf0.at[pl.ds(0, rem)],
                    acc.at[pl.ds(stripe + ROWS_PER_TILE - rem, rem)])

    plsc.subcore_barrier()

    bufs = (buf0, buf1)
    cidxs = (cidx0, cidx1)
    sems = (sem0, sem1)

    for phase in range(NPHASE):
        base = ebase + phase * EPP
        pltpu.sync_copy(rows_hbm.at[pl.ds(base, EPP)], row_v)
        pltpu.sync_copy(cols_hbm.at[pl.ds(base, EPP)], col_v)

        pltpu.async_copy(neigh_hbm.at[row_v.at[pl.ds(0, CHUNK)]], buf0, sem0)
        for l in range(CHUNK // 16):
            cidx0[0, pl.ds(l * 16, 16)] = col_v[pl.ds(l * 16, 16)]

        @pl.loop(0, CPP, step=2)
        def _chunks(g):
            for b in range(2):
                j = g + b
                @pl.when(j + 1 < CPP)
                def _():
                    pltpu.async_copy(
                        neigh_hbm.at[row_v.at[pl.ds((j + 1) * CHUNK, CHUNK)]],
                        bufs[1 - b], sems[1 - b])
                    for l in range(CHUNK // 16):
                        cidxs[1 - b][0, pl.ds(l * 16, 16)] = (
                            col_v[pl.ds((j + 1) * CHUNK + l * 16, 16)])
                pltpu.make_async_copy(
                    neigh_hbm.at[row_v.at[pl.ds(0, CHUNK)]],
                    bufs[b], sems[b]).wait()
                pltpu.sync_copy(bufs[b], acc.at[cidxs[b].at[0]], add=True)

    tbase = ebase + NPHASE * EPP
    pltpu.sync_copy(rows_hbm.at[pl.ds(tbase, TAIL)], row_v.at[pl.ds(0, TAIL)])
    pltpu.sync_copy(cols_hbm.at[pl.ds(tbase, TAIL)], col_v.at[pl.ds(0, TAIL)])
    tidx[0, :] = col_v[pl.ds(0, TAIL)]
    pltpu.async_copy(neigh_hbm.at[row_v.at[pl.ds(0, TAIL)]],
                     buf0.at[pl.ds(0, TAIL)], sem0)
    pltpu.make_async_copy(neigh_hbm.at[row_v.at[pl.ds(0, TAIL)]],
                          buf0.at[pl.ds(0, TAIL)], sem0).wait()
    pltpu.sync_copy(buf0.at[pl.ds(0, TAIL)], acc.at[tidx.at[0]], add=True)

    plsc.subcore_barrier()

    pltpu.sync_copy(acc.at[pl.ds(stripe, ROWS_PER_TILE)],
                    out_hbm.at[cid, pl.ds(stripe, ROWS_PER_TILE)])


_BLK = 1000
_DN = (((1,), (1,)), ((), ()))


def _mm_body(x_ref, w_ref, b_ref, o_ref):
    o_ref[...] = lax.dot_general(
        x_ref[...], w_ref[...], _DN, preferred_element_type=jnp.float32
    ) + b_ref[...]


def _addrelu_body(selfp_ref, p0_ref, p1_ref, o_ref):
    o_ref[...] = jnp.maximum(
        selfp_ref[...] + p0_ref[0, :, :] + p1_ref[0, :, :], 0.0)


def _matmul_bias(x, W, b):
    nblk = N_NODES // _BLK
    return pl.pallas_call(
        _mm_body,
        grid=(nblk,),
        in_specs=[
            pl.BlockSpec((_BLK, D), lambda i: (i, 0)),
            pl.BlockSpec((D, D), lambda i: (0, 0)),
            pl.BlockSpec((1, D), lambda i: (0, 0)),
        ],
        out_specs=pl.BlockSpec((_BLK, D), lambda i: (i, 0)),
        out_shape=jax.ShapeDtypeStruct((N_NODES, D), jnp.float32),
    )(x, W, b.reshape(1, D))


def kernel(x, edge_index, W_self, b_self, W_neigh, b_neigh):
    eidx = edge_index.astype(jnp.int32)

    neigh = _matmul_bias(x, W_neigh, b_neigh)
    partials = _sc_aggregate(neigh, eidx[0], eidx[1])
    selfp = _matmul_bias(x, W_self, b_self)

    nblk = N_NODES // _BLK
    out = pl.pallas_call(
        _addrelu_body,
        grid=(nblk,),
        in_specs=[
            pl.BlockSpec((_BLK, D), lambda i: (i, 0)),
            pl.BlockSpec((1, _BLK, D), lambda i: (0, i, 0)),
            pl.BlockSpec((1, _BLK, D), lambda i: (1, i, 0)),
        ],
        out_specs=pl.BlockSpec((_BLK, D), lambda i: (i, 0)),
        out_shape=jax.ShapeDtypeStruct((N_NODES, D), jnp.float32),
    )(selfp, partials, partials)

    return out


# speedup vs baseline: 1.2244x; 1.1275x over previous
import functools

import jax
import jax.numpy as jnp
from jax import lax
from jax.experimental import pallas as pl
from jax.experimental.pallas import tpu as pltpu
from jax.experimental.pallas import tpu_sc as plsc

N_NODES = 10000
N_EDGES = 320000
D = 128

NC = 2
NS = 16
NW = NC * NS
BLK_E = 128
NBLK = N_EDGES // BLK_E
NB_BASE = NBLK // NW
NB_EXTRA = NBLK % NW
NB_MAX = NB_BASE + (1 if NB_EXTRA else 0)
NT = (NB_MAX + 1) // 2
CHUNK = 64
ROWS_PER_TILE = 632
ACC_ROWS = NS * ROWS_PER_TILE
ZCH = 40

_mesh = plsc.VectorSubcoreMesh(core_axis_name="c", subcore_axis_name="s")


@functools.partial(
    pl.kernel,
    out_type=jax.ShapeDtypeStruct((NC, ACC_ROWS, D), jnp.float32),
    mesh=_mesh,
    scratch_types=[
        pltpu.VMEM((2, BLK_E), jnp.int32),
        pltpu.VMEM((2, BLK_E), jnp.int32),
        pltpu.VMEM((1, CHUNK), jnp.int32),
        pltpu.VMEM((1, CHUNK), jnp.int32),
        pltpu.VMEM((CHUNK, D), jnp.float32),
        pltpu.VMEM((CHUNK, D), jnp.float32),
        pltpu.VMEM_SHARED((ACC_ROWS, D), jnp.float32),
        pltpu.SemaphoreType.DMA,
        pltpu.SemaphoreType.DMA,
        pltpu.SemaphoreType.DMA,
        pltpu.SemaphoreType.DMA,
        pltpu.SemaphoreType.DMA,
        pltpu.SemaphoreType.DMA,
    ],
)
def _sc_aggregate(neigh_hbm, eidx_hbm, out_hbm,
                  ibuf0, ibuf1, cidx0, cidx1, db0, db1, acc,
                  isem0, isem1, gsem0, gsem1, ssem0, ssem1):
    cid = lax.axis_index("c")
    sid = lax.axis_index("s")
    wid = cid * NS + sid
    nb = NB_BASE + jnp.where(wid < NB_EXTRA, 1, 0)
    nch = 2 * nb

    ibufs = (ibuf0, ibuf1)
    cidxs = (cidx0, cidx1)
    dbs = (db0, db1)
    isems = (isem0, isem1)
    gsems = (gsem0, gsem1)
    ssems = (ssem0, ssem1)

    def idx_copy(blk_local, ib):
        off = pl.multiple_of((wid + NW * blk_local) * BLK_E, BLK_E)
        pltpu.async_copy(eidx_hbm.at[:, pl.ds(off, BLK_E)], ibufs[ib], isems[ib])

    def idx_wait(ib):
        pltpu.make_async_copy(
            eidx_hbm.at[:, pl.ds(0, BLK_E)], ibufs[ib], isems[ib]).wait()

    def repack(ib, h, cp):
        for l in range(CHUNK // 16):
            cidxs[cp][0, pl.ds(l * 16, 16)] = (
                ibufs[ib][1, pl.ds(h * CHUNK + l * 16, 16)])

    def gather_start(ib, h, p):
        pltpu.async_copy(
            neigh_hbm.at[ibufs[ib].at[0, pl.ds(h * CHUNK, CHUNK)]],
            dbs[p], gsems[p])

    def gather_wait(p):
        pltpu.make_async_copy(
            neigh_hbm.at[ibufs[0].at[0, pl.ds(0, CHUNK)]],
            dbs[p], gsems[p]).wait()

    def scatter_start(p):
        pltpu.async_copy(dbs[p], acc.at[cidxs[p].at[0]], ssems[p], add=True)

    def scatter_wait(p):
        pltpu.make_async_copy(
            dbs[p], acc.at[cidxs[p].at[0]], ssems[p]).wait()

    zero16 = jnp.zeros((16,), jnp.float32)

    @pl.loop(0, ZCH)
    def _zero_rows(r):
        for j in range(D // 16):
            db0[r, pl.ds(j * 16, 16)] = zero16

    stripe = sid * ROWS_PER_TILE
    for t in range(ROWS_PER_TILE // ZCH):
        pltpu.sync_copy(db0.at[pl.ds(0, ZCH)],
                        acc.at[pl.ds(stripe + t * ZCH, ZCH)])
    rem = ROWS_PER_TILE % ZCH
    pltpu.sync_copy(db0.at[pl.ds(0, rem)],
                    acc.at[pl.ds(stripe + ROWS_PER_TILE - rem, rem)])

    idx_copy(0, 0)
    idx_copy(1, 1)

    plsc.subcore_barrier()

    idx_wait(0)
    repack(0, 0, 0)
    gather_start(0, 0, 0)

    @pl.loop(0, NT)
    def _pairs(t):
        for bb in range(2):
            for h in range(2):
                blk = 2 * t + bb
                c = 2 * blk + h
                p = h

                @pl.when(c + 1 < nch)
                def _():
                    @pl.when(c >= 1)
                    def _():
                        scatter_wait(1 - p)
                    if h == 0:
                        repack(bb, 1, 1 - p)
                        gather_start(bb, 1, 1 - p)
                    else:
                        idx_wait(1 - bb)
                        repack(1 - bb, 0, 1 - p)
                        gather_start(1 - bb, 0, 1 - p)
                        @pl.when(blk + 2 < nb)
                        def _():
                            idx_copy(blk + 2, bb)

                @pl.when(c < nch)
                def _():
                    gather_wait(p)
                    scatter_start(p)

    scatter_wait(0)
    scatter_wait(1)

    plsc.subcore_barrier()

    pltpu.sync_copy(acc.at[pl.ds(stripe, ROWS_PER_TILE)],
                    out_hbm.at[cid, pl.ds(stripe, ROWS_PER_TILE)])


_BLK = 1000
_DN = (((1,), (1,)), ((), ()))


def _mm_body(x_ref, w_ref, b_ref, o_ref):
    o_ref[...] = lax.dot_general(
        x_ref[...], w_ref[...], _DN, preferred_element_type=jnp.float32
    ) + b_ref[...]


def _addrelu_body(selfp_ref, p0_ref, p1_ref, o_ref):
    o_ref[...] = jnp.maximum(
        selfp_ref[...] + p0_ref[0, :, :] + p1_ref[0, :, :], 0.0)


def _matmul_bias(x, W, b):
    nblk = N_NODES // _BLK
    return pl.pallas_call(
        _mm_body,
        grid=(nblk,),
        in_specs=[
            pl.BlockSpec((_BLK, D), lambda i: (i, 0)),
            pl.BlockSpec((D, D), lambda i: (0, 0)),
            pl.BlockSpec((1, D), lambda i: (0, 0)),
        ],
        out_specs=pl.BlockSpec((_BLK, D), lambda i: (i, 0)),
        out_shape=jax.ShapeDtypeStruct((N_NODES, D), jnp.float32),
    )(x, W, b.reshape(1, D))


def kernel(x, edge_index, W_self, b_self, W_neigh, b_neigh):
    eidx = edge_index.astype(jnp.int32)

    neigh = _matmul_bias(x, W_neigh, b_neigh)
    partials = _sc_aggregate(neigh, eidx)
    selfp = _matmul_bias(x, W_self, b_self)

    nblk = N_NODES // _BLK
    out = pl.pallas_call(
        _addrelu_body,
        grid=(nblk,),
        in_specs=[
            pl.BlockSpec((_BLK, D), lambda i: (i, 0)),
            pl.BlockSpec((1, _BLK, D), lambda i: (0, i, 0)),
            pl.BlockSpec((1, _BLK, D), lambda i: (1, i, 0)),
        ],
        out_specs=pl.BlockSpec((_BLK, D), lambda i: (i, 0)),
        out_shape=jax.ShapeDtypeStruct((N_NODES, D), jnp.float32),
    )(selfp, partials, partials)

    return out


# speedup vs baseline: 1.4883x; 1.2156x over previous
import functools

import jax
import jax.numpy as jnp
from jax import lax
from jax.experimental import pallas as pl
from jax.experimental.pallas import tpu as pltpu
from jax.experimental.pallas import tpu_sc as plsc

N_NODES = 10000
N_EDGES = 320000
D = 128

NC = 2
NS = 16
NW = NC * NS
BLK_E = 128
NBLK = N_EDGES // BLK_E
NB_BASE = NBLK // NW
NB_EXTRA = NBLK % NW
NB_MAX = NB_BASE + (1 if NB_EXTRA else 0)
NT = (NB_MAX + 2) // 3
CHUNK = 64
ROWS_PER_TILE = 632
ACC_ROWS = NS * ROWS_PER_TILE
ZCH = 40

_mesh = plsc.VectorSubcoreMesh(core_axis_name="c", subcore_axis_name="s")


@functools.partial(
    pl.kernel,
    out_type=jax.ShapeDtypeStruct((NC, ACC_ROWS, D), jnp.float32),
    mesh=_mesh,
    scratch_types=[
        pltpu.VMEM((2, BLK_E), jnp.int32),
        pltpu.VMEM((2, BLK_E), jnp.int32),
        pltpu.VMEM((2, BLK_E), jnp.int32),
        pltpu.VMEM((1, CHUNK), jnp.int32),
        pltpu.VMEM((1, CHUNK), jnp.int32),
        pltpu.VMEM((1, CHUNK), jnp.int32),
        pltpu.VMEM((CHUNK, D), jnp.float32),
        pltpu.VMEM((CHUNK, D), jnp.float32),
        pltpu.VMEM((CHUNK, D), jnp.float32),
        pltpu.VMEM_SHARED((ACC_ROWS, D), jnp.float32),
        pltpu.SemaphoreType.DMA,
        pltpu.SemaphoreType.DMA,
        pltpu.SemaphoreType.DMA,
        pltpu.SemaphoreType.DMA,
        pltpu.SemaphoreType.DMA,
        pltpu.SemaphoreType.DMA,
        pltpu.SemaphoreType.DMA,
        pltpu.SemaphoreType.DMA,
        pltpu.SemaphoreType.DMA,
    ],
)
def _sc_aggregate(neigh_hbm, eidx_hbm, out_hbm,
                  ibuf0, ibuf1, ibuf2, cidx0, cidx1, cidx2, db0, db1, db2, acc,
                  isem0, isem1, isem2, gsem0, gsem1, gsem2,
                  ssem0, ssem1, ssem2):
    cid = lax.axis_index("c")
    sid = lax.axis_index("s")
    wid = cid * NS + sid
    nb = NB_BASE + jnp.where(wid < NB_EXTRA, 1, 0)
    nch = 2 * nb

    ibufs = (ibuf0, ibuf1, ibuf2)
    cidxs = (cidx0, cidx1, cidx2)
    dbs = (db0, db1, db2)
    isems = (isem0, isem1, isem2)
    gsems = (gsem0, gsem1, gsem2)
    ssems = (ssem0, ssem1, ssem2)

    def idx_copy(blk_local, ib):
        off = pl.multiple_of((wid + NW * blk_local) * BLK_E, BLK_E)
        pltpu.async_copy(eidx_hbm.at[:, pl.ds(off, BLK_E)], ibufs[ib], isems[ib])

    def idx_wait(ib):
        pltpu.make_async_copy(
            eidx_hbm.at[:, pl.ds(0, BLK_E)], ibufs[ib], isems[ib]).wait()

    def repack(ib, h, cp):
        for l in range(CHUNK // 16):
            cidxs[cp][0, pl.ds(l * 16, 16)] = (
                ibufs[ib][1, pl.ds(h * CHUNK + l * 16, 16)])

    def gather_start(ib, h, p):
        pltpu.async_copy(
            neigh_hbm.at[ibufs[ib].at[0, pl.ds(h * CHUNK, CHUNK)]],
            dbs[p], gsems[p])

    def gather_wait(p):
        pltpu.make_async_copy(
            neigh_hbm.at[ibufs[0].at[0, pl.ds(0, CHUNK)]],
            dbs[p], gsems[p]).wait()

    def scatter_start(p):
        pltpu.async_copy(dbs[p], acc.at[cidxs[p].at[0]], ssems[p], add=True)

    def scatter_wait(p):
        pltpu.make_async_copy(
            dbs[p], acc.at[cidxs[p].at[0]], ssems[p]).wait()

    zero16 = jnp.zeros((16,), jnp.float32)

    @pl.loop(0, ZCH)
    def _zero_rows(r):
        for j in range(D // 16):
            db0[r, pl.ds(j * 16, 16)] = zero16

    stripe = sid * ROWS_PER_TILE
    for t in range(ROWS_PER_TILE // ZCH):
        pltpu.sync_copy(db0.at[pl.ds(0, ZCH)],
                        acc.at[pl.ds(stripe + t * ZCH, ZCH)])
    rem = ROWS_PER_TILE % ZCH
    pltpu.sync_copy(db0.at[pl.ds(0, rem)],
                    acc.at[pl.ds(stripe + ROWS_PER_TILE - rem, rem)])

    idx_copy(0, 0)
    idx_copy(1, 1)
    idx_copy(2, 2)

    plsc.subcore_barrier()

    idx_wait(0)
    repack(0, 0, 0)
    gather_start(0, 0, 0)
    repack(0, 1, 1)
    gather_start(0, 1, 1)

    @pl.loop(0, NT)
    def _triples(t):
        for u in range(3):
            for h in range(2):
                blk = 3 * t + u
                c = 2 * blk + h
                p = (2 * u + h) % 3
                q = (2 * u + h + 2) % 3
                ib2 = (u + 1) % 3

                @pl.when(c + 2 < nch)
                def _():
                    @pl.when(c >= 1)
                    def _():
                        scatter_wait(q)
                    if h == 0:
                        idx_wait(ib2)
                    repack(ib2, h, q)
                    gather_start(ib2, h, q)

                @pl.when(c < nch)
                def _():
                    gather_wait(p)
                    scatter_start(p)
                    if h == 1:
                        @pl.when(blk + 3 < nb)
                        def _():
                            idx_copy(blk + 3, u)

    scatter_wait(0)
    scatter_wait(1)
    scatter_wait(2)

    plsc.subcore_barrier()

    pltpu.sync_copy(acc.at[pl.ds(stripe, ROWS_PER_TILE)],
                    out_hbm.at[cid, pl.ds(stripe, ROWS_PER_TILE)])


_BLK = 1000
_DN = (((1,), (1,)), ((), ()))


def _mm_body(x_ref, w_ref, b_ref, o_ref):
    o_ref[...] = lax.dot_general(
        x_ref[...], w_ref[...], _DN, preferred_element_type=jnp.float32
    ) + b_ref[...]


def _addrelu_body(selfp_ref, p0_ref, p1_ref, o_ref):
    o_ref[...] = jnp.maximum(
        selfp_ref[...] + p0_ref[0, :, :] + p1_ref[0, :, :], 0.0)


def _matmul_bias(x, W, b):
    nblk = N_NODES // _BLK
    return pl.pallas_call(
        _mm_body,
        grid=(nblk,),
        in_specs=[
            pl.BlockSpec((_BLK, D), lambda i: (i, 0)),
            pl.BlockSpec((D, D), lambda i: (0, 0)),
            pl.BlockSpec((1, D), lambda i: (0, 0)),
        ],
        out_specs=pl.BlockSpec((_BLK, D), lambda i: (i, 0)),
        out_shape=jax.ShapeDtypeStruct((N_NODES, D), jnp.float32),
    )(x, W, b.reshape(1, D))


def kernel(x, edge_index, W_self, b_self, W_neigh, b_neigh):
    eidx = edge_index.astype(jnp.int32)

    neigh = _matmul_bias(x, W_neigh, b_neigh)
    partials = _sc_aggregate(neigh, eidx)
    selfp = _matmul_bias(x, W_self, b_self)

    nblk = N_NODES // _BLK
    out = pl.pallas_call(
        _addrelu_body,
        grid=(nblk,),
        in_specs=[
            pl.BlockSpec((_BLK, D), lambda i: (i, 0)),
            pl.BlockSpec((1, _BLK, D), lambda i: (0, i, 0)),
            pl.BlockSpec((1, _BLK, D), lambda i: (1, i, 0)),
        ],
        out_specs=pl.BlockSpec((_BLK, D), lambda i: (i, 0)),
        out_shape=jax.ShapeDtypeStruct((N_NODES, D), jnp.float32),
    )(selfp, partials, partials)

    return out


# speedup vs baseline: 1.5421x; 1.0362x over previous
import functools

import jax
import jax.numpy as jnp
from jax import lax
from jax.experimental import pallas as pl
from jax.experimental.pallas import tpu as pltpu
from jax.experimental.pallas import tpu_sc as plsc

N_NODES = 10000
N_EDGES = 320000
D = 128

NC = 2
NS = 16
NW = NC * NS
BLK_E = 128
NBLK = N_EDGES // BLK_E
NB_BASE = NBLK // NW
NB_EXTRA = NBLK % NW
NB_MAX = NB_BASE + (1 if NB_EXTRA else 0)
NT = (NB_MAX + 2) // 3
CHUNK = 64
ROWS_PER_TILE = 632
ACC_ROWS = NS * ROWS_PER_TILE
ZCH = 40

_mesh = plsc.VectorSubcoreMesh(core_axis_name="c", subcore_axis_name="s")


@functools.partial(
    pl.kernel,
    out_type=jax.ShapeDtypeStruct((NC, ACC_ROWS, D), jnp.float32),
    mesh=_mesh,
    scratch_types=[
        pltpu.VMEM((2, BLK_E), jnp.int32),
        pltpu.VMEM((2, BLK_E), jnp.int32),
        pltpu.VMEM((2, BLK_E), jnp.int32),
        pltpu.VMEM((1, CHUNK), jnp.int32),
        pltpu.VMEM((1, CHUNK), jnp.int32),
        pltpu.VMEM((1, CHUNK), jnp.int32),
        pltpu.VMEM((CHUNK, D), jnp.float32),
        pltpu.VMEM((CHUNK, D), jnp.float32),
        pltpu.VMEM((CHUNK, D), jnp.float32),
        pltpu.VMEM_SHARED((ACC_ROWS, D), jnp.float32),
        pltpu.SemaphoreType.DMA,
        pltpu.SemaphoreType.DMA,
        pltpu.SemaphoreType.DMA,
        pltpu.SemaphoreType.DMA,
        pltpu.SemaphoreType.DMA,
        pltpu.SemaphoreType.DMA,
        pltpu.SemaphoreType.DMA,
        pltpu.SemaphoreType.DMA,
        pltpu.SemaphoreType.DMA,
    ],
)
def _sc_aggregate(neigh_hbm, eidx_hbm, out_hbm,
                  ibuf0, ibuf1, ibuf2, cidx0, cidx1, cidx2, db0, db1, db2, acc,
                  isem0, isem1, isem2, gsem0, gsem1, gsem2,
                  ssem0, ssem1, ssem2):
    cid = lax.axis_index("c")
    sid = lax.axis_index("s")
    wid = cid * NS + sid
    nb = NB_BASE + jnp.where(wid < NB_EXTRA, 1, 0)
    nch = 2 * nb

    ibufs = (ibuf0, ibuf1, ibuf2)
    cidxs = (cidx0, cidx1, cidx2)
    dbs = (db0, db1, db2)
    isems = (isem0, isem1, isem2)
    gsems = (gsem0, gsem1, gsem2)
    ssems = (ssem0, ssem1, ssem2)

    def idx_copy(blk_local, ib):
        off = pl.multiple_of((wid + NW * blk_local) * BLK_E, BLK_E)
        pltpu.async_copy(eidx_hbm.at[:, pl.ds(off, BLK_E)], ibufs[ib], isems[ib])

    def idx_wait(ib):
        pltpu.make_async_copy(
            eidx_hbm.at[:, pl.ds(0, BLK_E)], ibufs[ib], isems[ib]).wait()

    def repack(ib, h, cp):
        for l in range(CHUNK // 16):
            cidxs[cp][0, pl.ds(l * 16, 16)] = (
                ibufs[ib][1, pl.ds(h * CHUNK + l * 16, 16)])

    def gather_start(ib, h, p):
        pltpu.async_copy(
            neigh_hbm.at[ibufs[ib].at[0, pl.ds(h * CHUNK, CHUNK)]],
            dbs[p], gsems[p])

    def gather_wait(p):
        pltpu.make_async_copy(
            neigh_hbm.at[ibufs[0].at[0, pl.ds(0, CHUNK)]],
            dbs[p], gsems[p]).wait()

    def scatter_start(p):
        pltpu.async_copy(dbs[p], acc.at[cidxs[p].at[0]], ssems[p], add=True)

    def scatter_wait(p):
        pltpu.make_async_copy(
            dbs[p], acc.at[cidxs[p].at[0]], ssems[p]).wait()

    zero16 = jnp.zeros((16,), jnp.float32)

    @pl.loop(0, ZCH)
    def _zero_rows(r):
        for j in range(D // 16):
            db0[r, pl.ds(j * 16, 16)] = zero16

    stripe = sid * ROWS_PER_TILE
    for t in range(ROWS_PER_TILE // ZCH):
        pltpu.sync_copy(db0.at[pl.ds(0, ZCH)],
                        acc.at[pl.ds(stripe + t * ZCH, ZCH)])
    rem = ROWS_PER_TILE % ZCH
    pltpu.sync_copy(db0.at[pl.ds(0, rem)],
                    acc.at[pl.ds(stripe + ROWS_PER_TILE - rem, rem)])

    idx_copy(0, 0)
    idx_copy(1, 1)
    idx_copy(2, 2)

    plsc.subcore_barrier()

    idx_wait(0)
    repack(0, 0, 0)
    gather_start(0, 0, 0)
    repack(0, 1, 1)
    gather_start(0, 1, 1)

    @pl.loop(0, NT)
    def _triples(t):
        for u in range(3):
            for h in range(2):
                blk = 3 * t + u
                c = 2 * blk + h
                p = (2 * u + h) % 3
                q = (2 * u + h + 2) % 3
                ib2 = (u + 1) % 3

                @pl.when(c + 2 < nch)
                def _():
                    @pl.when(c >= 1)
                    def _():
                        scatter_wait(q)
                    if h == 0:
                        idx_wait(ib2)
                    repack(ib2, h, q)
                    gather_start(ib2, h, q)

                @pl.when(c < nch)
                def _():
                    gather_wait(p)
                    scatter_start(p)
                    if h == 1:
                        @pl.when(blk + 3 < nb)
                        def _():
                            idx_copy(blk + 3, u)

    scatter_wait(0)
    scatter_wait(1)
    scatter_wait(2)

    plsc.subcore_barrier()

    pltpu.sync_copy(acc.at[pl.ds(stripe, ROWS_PER_TILE)],
                    out_hbm.at[cid, pl.ds(stripe, ROWS_PER_TILE)])


_BLK = 2000
_DN = (((1,), (1,)), ((), ()))


def _mm_body(x_ref, w_ref, b_ref, o_ref):
    o_ref[...] = lax.dot_general(
        x_ref[...], w_ref[...], _DN, preferred_element_type=jnp.float32
    ) + b_ref[...]


def _addrelu_body(selfp_ref, p0_ref, p1_ref, o_ref):
    o_ref[...] = jnp.maximum(
        selfp_ref[...] + p0_ref[0, :, :] + p1_ref[0, :, :], 0.0)


def _matmul_bias(x, W, b):
    nblk = N_NODES // _BLK
    return pl.pallas_call(
        _mm_body,
        grid=(nblk,),
        in_specs=[
            pl.BlockSpec((_BLK, D), lambda i: (i, 0)),
            pl.BlockSpec((D, D), lambda i: (0, 0)),
            pl.BlockSpec((1, D), lambda i: (0, 0)),
        ],
        out_specs=pl.BlockSpec((_BLK, D), lambda i: (i, 0)),
        out_shape=jax.ShapeDtypeStruct((N_NODES, D), jnp.float32),
    )(x, W, b.reshape(1, D))


def kernel(x, edge_index, W_self, b_self, W_neigh, b_neigh):
    eidx = edge_index.astype(jnp.int32)

    neigh = _matmul_bias(x, W_neigh, b_neigh)
    partials = _sc_aggregate(neigh, eidx)
    selfp = _matmul_bias(x, W_self, b_self)

    nblk = N_NODES // _BLK
    out = pl.pallas_call(
        _addrelu_body,
        grid=(nblk,),
        in_specs=[
            pl.BlockSpec((_BLK, D), lambda i: (i, 0)),
            pl.BlockSpec((1, _BLK, D), lambda i: (0, i, 0)),
            pl.BlockSpec((1, _BLK, D), lambda i: (1, i, 0)),
        ],
        out_specs=pl.BlockSpec((_BLK, D), lambda i: (i, 0)),
        out_shape=jax.ShapeDtypeStruct((N_NODES, D), jnp.float32),
    )(selfp, partials, partials)

    return out
